# baseline (speedup 1.0000x reference)
import jax
import jax.numpy as jnp
from jax.experimental import pallas as pl
from jax.experimental.pallas import tpu as pltpu

DIMS = 1024
E = 8
BT = 512


def _body(x_ref, w_ref, bg_ref, wc_ref, bc_ref, a_ref, o_ref):
    x = x_ref[...]
    lc = jnp.dot(x, wc_ref[...], preferred_element_type=jnp.float32) + bc_ref[...]
    lt = lc[:, :E]
    lf = lc[:, E:]
    iota = jax.lax.broadcasted_iota(jnp.int32, lt.shape, 1)
    m1 = jnp.max(lt, axis=-1, keepdims=True)
    i1 = jnp.min(jnp.where(lt == m1, iota, E), axis=-1, keepdims=True)
    masked = jnp.where(iota == i1, -jnp.inf, lt)
    m2 = jnp.max(masked, axis=-1, keepdims=True)
    i2 = jnp.min(jnp.where(masked == m2, iota, E), axis=-1, keepdims=True)
    t = jnp.exp(m2 - m1)
    w1 = 1.0 / (1.0 + t)
    w2 = t / (1.0 + t)
    type_ = jnp.where(iota == i1, w1, 0.0) + jnp.where(iota == i2, w2, 0.0)
    lf = lf - jnp.max(lf, axis=-1, keepdims=True)
    ef = jnp.exp(lf)
    feat = ef / jnp.sum(ef, axis=-1, keepdims=True)
    a = a_ref[0, 0]
    hfeats = (0.5 * a) * type_ + (0.5 * (1.0 - a)) * feat
    xb = (0.5 * x).astype(jnp.bfloat16)
    acc = jnp.sum(hfeats, axis=-1, keepdims=True)
    for e in range(E):
        z = jnp.dot(xb, w_ref[e], preferred_element_type=jnp.float32)
        th = jnp.tanh(z + bg_ref[e])
        acc = acc + th * hfeats[:, e:e + 1]
    o_ref[...] = acc


def kernel(x, W_gates, b_gates, W_f, b_f, W_t, b_t, alpha, num):
    B, S, D = x.shape
    M = B * S
    xf = x.reshape(M, D)
    a = jax.nn.sigmoid(alpha).reshape(1, 1).astype(jnp.float32)
    Wb = W_gates.astype(jnp.bfloat16)
    Wc = jnp.concatenate([W_t, W_f], axis=1)
    bc = jnp.concatenate([b_t, b_f]).reshape(1, 2 * E)
    hbg = 0.5 * b_gates

    out = pl.pallas_call(
        _body,
        grid=(M // BT,),
        in_specs=[
            pl.BlockSpec((BT, D), lambda t: (t, 0)),
            pl.BlockSpec((E, D, D), lambda t: (0, 0, 0)),
            pl.BlockSpec((E, D), lambda t: (0, 0)),
            pl.BlockSpec((D, 2 * E), lambda t: (0, 0)),
            pl.BlockSpec((1, 2 * E), lambda t: (0, 0)),
            pl.BlockSpec((1, 1), lambda t: (0, 0)),
        ],
        out_specs=pl.BlockSpec((BT, D), lambda t: (t, 0)),
        out_shape=jax.ShapeDtypeStruct((M, D), jnp.float32),
        compiler_params=pltpu.CompilerParams(
            dimension_semantics=("parallel",),
        ),
    )(xf, Wb, hbg, Wc, bc, a)

    return out.reshape(B, S, D)


# speedup vs baseline: 1.0240x; 1.0240x over previous
import jax
import jax.numpy as jnp
from jax.experimental import pallas as pl
from jax.experimental.pallas import tpu as pltpu

DIMS = 1024
E = 8
BT = 1024
NCH = 2


def _body(x_ref, w_ref, bg_ref, wc_ref, bc_ref, a_ref, o_ref):
    x = x_ref[...]
    lc = jnp.dot(x, wc_ref[...], preferred_element_type=jnp.float32) + bc_ref[...]
    lt = lc[:, :E]
    lf = lc[:, E:]
    iota = jax.lax.broadcasted_iota(jnp.int32, lt.shape, 1)
    m1 = jnp.max(lt, axis=-1, keepdims=True)
    i1 = jnp.min(jnp.where(lt == m1, iota, E), axis=-1, keepdims=True)
    masked = jnp.where(iota == i1, -jnp.inf, lt)
    m2 = jnp.max(masked, axis=-1, keepdims=True)
    i2 = jnp.min(jnp.where(masked == m2, iota, E), axis=-1, keepdims=True)
    t = jnp.exp(m2 - m1)
    w1 = 1.0 / (1.0 + t)
    w2 = t / (1.0 + t)
    type_ = jnp.where(iota == i1, w1, 0.0) + jnp.where(iota == i2, w2, 0.0)
    lf = lf - jnp.max(lf, axis=-1, keepdims=True)
    ef = jnp.exp(lf)
    feat = ef / jnp.sum(ef, axis=-1, keepdims=True)
    a = a_ref[0, 0]
    hfeats = (0.5 * a) * type_ + (0.5 * (1.0 - a)) * feat
    xb = (0.5 * x).astype(jnp.bfloat16)
    hfsum = jnp.sum(hfeats, axis=-1, keepdims=True)
    C = DIMS // NCH
    for n in range(NCH):
        sl = slice(n * C, (n + 1) * C)
        acc = jnp.broadcast_to(hfsum, (xb.shape[0], C))
        for e in range(E):
            z = jnp.dot(xb, w_ref[e][:, sl], preferred_element_type=jnp.float32)
            acc = acc + jnp.tanh(z + bg_ref[e][sl]) * hfeats[:, e:e + 1]
        o_ref[:, sl] = acc


def kernel(x, W_gates, b_gates, W_f, b_f, W_t, b_t, alpha, num):
    B, S, D = x.shape
    M = B * S
    xf = x.reshape(M, D)
    a = jax.nn.sigmoid(alpha).reshape(1, 1).astype(jnp.float32)
    Wb = W_gates.astype(jnp.bfloat16)
    Wc = jnp.concatenate([W_t, W_f], axis=1)
    bc = jnp.concatenate([b_t, b_f]).reshape(1, 2 * E)
    hbg = 0.5 * b_gates

    out = pl.pallas_call(
        _body,
        grid=(M // BT,),
        in_specs=[
            pl.BlockSpec((BT, D), lambda t: (t, 0)),
            pl.BlockSpec((E, D, D), lambda t: (0, 0, 0)),
            pl.BlockSpec((E, D), lambda t: (0, 0)),
            pl.BlockSpec((D, 2 * E), lambda t: (0, 0)),
            pl.BlockSpec((1, 2 * E), lambda t: (0, 0)),
            pl.BlockSpec((1, 1), lambda t: (0, 0)),
        ],
        out_specs=pl.BlockSpec((BT, D), lambda t: (t, 0)),
        out_shape=jax.ShapeDtypeStruct((M, D), jnp.float32),
        compiler_params=pltpu.CompilerParams(
            dimension_semantics=("parallel",),
        ),
    )(xf, Wb, hbg, Wc, bc, a)

    return out.reshape(B, S, D)


# speedup vs baseline: 1.1313x; 1.1047x over previous
import jax
import jax.numpy as jnp
from jax.experimental import pallas as pl
from jax.experimental.pallas import tpu as pltpu

DIMS = 1024
E = 8
BT = 512
NCH = 4


def _body(x_ref, w_ref, bg_ref, wc_ref, bc_ref, a_ref, o_ref):
    x = x_ref[...]
    lc = jnp.dot(x, wc_ref[...], preferred_element_type=jnp.float32) + bc_ref[...]
    lt = lc[:, :E]
    lf = lc[:, E:]
    iota = jax.lax.broadcasted_iota(jnp.int32, lt.shape, 1)
    m1 = jnp.max(lt, axis=-1, keepdims=True)
    i1 = jnp.min(jnp.where(lt == m1, iota, E), axis=-1, keepdims=True)
    masked = jnp.where(iota == i1, -jnp.inf, lt)
    m2 = jnp.max(masked, axis=-1, keepdims=True)
    i2 = jnp.min(jnp.where(masked == m2, iota, E), axis=-1, keepdims=True)
    t = jnp.exp(m2 - m1)
    w1 = 1.0 / (1.0 + t)
    w2 = t / (1.0 + t)
    type_ = jnp.where(iota == i1, w1, 0.0) + jnp.where(iota == i2, w2, 0.0)
    lf = lf - jnp.max(lf, axis=-1, keepdims=True)
    ef = jnp.exp(lf)
    feat = ef / jnp.sum(ef, axis=-1, keepdims=True)
    a = a_ref[0, 0]
    hfeats = (0.5 * a) * type_ + (0.5 * (1.0 - a)) * feat
    xb = 0.5 * x
    hfsum = jnp.sum(hfeats, axis=-1, keepdims=True)
    C = DIMS // NCH
    for n in range(NCH):
        sl = slice(n * C, (n + 1) * C)
        acc = jnp.broadcast_to(hfsum, (xb.shape[0], C))
        for e in range(E):
            z = jnp.dot(xb, w_ref[e][:, sl], preferred_element_type=jnp.float32)
            acc = acc + jnp.tanh(z + bg_ref[e][sl]) * hfeats[:, e:e + 1]
        o_ref[:, sl] = acc


def kernel(x, W_gates, b_gates, W_f, b_f, W_t, b_t, alpha, num):
    B, S, D = x.shape
    M = B * S
    xf = x.reshape(M, D)
    a = jax.nn.sigmoid(alpha).reshape(1, 1).astype(jnp.float32)
    Wb = W_gates
    Wc = jnp.concatenate([W_t, W_f], axis=1)
    bc = jnp.concatenate([b_t, b_f]).reshape(1, 2 * E)
    hbg = 0.5 * b_gates

    out = pl.pallas_call(
        _body,
        grid=(M // BT,),
        in_specs=[
            pl.BlockSpec((BT, D), lambda t: (t, 0)),
            pl.BlockSpec((E, D, D), lambda t: (0, 0, 0)),
            pl.BlockSpec((E, D), lambda t: (0, 0)),
            pl.BlockSpec((D, 2 * E), lambda t: (0, 0)),
            pl.BlockSpec((1, 2 * E), lambda t: (0, 0)),
            pl.BlockSpec((1, 1), lambda t: (0, 0)),
        ],
        out_specs=pl.BlockSpec((BT, D), lambda t: (t, 0)),
        out_shape=jax.ShapeDtypeStruct((M, D), jnp.float32),
        compiler_params=pltpu.CompilerParams(
            dimension_semantics=("parallel",),
        ),
    )(xf, Wb, hbg, Wc, bc, a)

    return out.reshape(B, S, D)


# speedup vs baseline: 1.1419x; 1.0094x over previous
import jax
import jax.numpy as jnp
from jax.experimental import pallas as pl
from jax.experimental.pallas import tpu as pltpu

DIMS = 1024
E = 8
BT = 512
NCH = 4


def _body(x_ref, w_ref, bg_ref, wc_ref, bc_ref, a_ref, o_ref):
    x = x_ref[...]
    lc = jnp.dot(x, wc_ref[...], preferred_element_type=jnp.float32) + bc_ref[...]
    lt = lc[:, :E]
    lf = lc[:, E:]
    iota = jax.lax.broadcasted_iota(jnp.int32, lt.shape, 1)
    m1 = jnp.max(lt, axis=-1, keepdims=True)
    i1 = jnp.min(jnp.where(lt == m1, iota, E), axis=-1, keepdims=True)
    masked = jnp.where(iota == i1, -jnp.inf, lt)
    m2 = jnp.max(masked, axis=-1, keepdims=True)
    i2 = jnp.min(jnp.where(masked == m2, iota, E), axis=-1, keepdims=True)
    t = jnp.exp(m2 - m1)
    w1 = 1.0 / (1.0 + t)
    w2 = t / (1.0 + t)
    type_ = jnp.where(iota == i1, w1, 0.0) + jnp.where(iota == i2, w2, 0.0)
    lf = lf - jnp.max(lf, axis=-1, keepdims=True)
    ef = jnp.exp(lf)
    feat = ef / jnp.sum(ef, axis=-1, keepdims=True)
    a = a_ref[0, 0]
    hfeats = (0.5 * a) * type_ + (0.5 * (1.0 - a)) * feat
    xb = (0.5 * x).astype(jnp.bfloat16)
    hfsum = jnp.sum(hfeats, axis=-1, keepdims=True)
    C = DIMS // NCH
    for n in range(NCH):
        sl = slice(n * C, (n + 1) * C)
        acc = jnp.broadcast_to(hfsum, (xb.shape[0], C))
        for e in range(E):
            z = jnp.dot(xb, w_ref[e][:, sl].astype(jnp.bfloat16), preferred_element_type=jnp.float32)
            acc = acc + jnp.tanh(z + bg_ref[e][sl]) * hfeats[:, e:e + 1]
        o_ref[:, sl] = acc


def kernel(x, W_gates, b_gates, W_f, b_f, W_t, b_t, alpha, num):
    B, S, D = x.shape
    M = B * S
    xf = x.reshape(M, D)
    a = jax.nn.sigmoid(alpha).reshape(1, 1).astype(jnp.float32)
    Wb = W_gates
    Wc = jnp.concatenate([W_t, W_f], axis=1)
    bc = jnp.concatenate([b_t, b_f]).reshape(1, 2 * E)
    hbg = 0.5 * b_gates

    out = pl.pallas_call(
        _body,
        grid=(M // BT,),
        in_specs=[
            pl.BlockSpec((BT, D), lambda t: (t, 0)),
            pl.BlockSpec((E, D, D), lambda t: (0, 0, 0)),
            pl.BlockSpec((E, D), lambda t: (0, 0)),
            pl.BlockSpec((D, 2 * E), lambda t: (0, 0)),
            pl.BlockSpec((1, 2 * E), lambda t: (0, 0)),
            pl.BlockSpec((1, 1), lambda t: (0, 0)),
        ],
        out_specs=pl.BlockSpec((BT, D), lambda t: (t, 0)),
        out_shape=jax.ShapeDtypeStruct((M, D), jnp.float32),
        compiler_params=pltpu.CompilerParams(
            dimension_semantics=("parallel",),
        ),
    )(xf, Wb, hbg, Wc, bc, a)

    return out.reshape(B, S, D)


# speedup vs baseline: 1.1789x; 1.0324x over previous
import jax
import jax.numpy as jnp
from jax.experimental import pallas as pl
from jax.experimental.pallas import tpu as pltpu

DIMS = 1024
E = 8
BT = 1024
NCH = 4


def _body(x_ref, w_ref, bg_ref, wc_ref, bc_ref, a_ref, o_ref):
    x = x_ref[...]
    lc = jnp.dot(x, wc_ref[...], preferred_element_type=jnp.float32) + bc_ref[...]
    lt = lc[:, :E]
    lf = lc[:, E:]
    iota = jax.lax.broadcasted_iota(jnp.int32, lt.shape, 1)
    m1 = jnp.max(lt, axis=-1, keepdims=True)
    i1 = jnp.min(jnp.where(lt == m1, iota, E), axis=-1, keepdims=True)
    masked = jnp.where(iota == i1, -jnp.inf, lt)
    m2 = jnp.max(masked, axis=-1, keepdims=True)
    i2 = jnp.min(jnp.where(masked == m2, iota, E), axis=-1, keepdims=True)
    t = jnp.exp(m2 - m1)
    w1 = 1.0 / (1.0 + t)
    w2 = t / (1.0 + t)
    type_ = jnp.where(iota == i1, w1, 0.0) + jnp.where(iota == i2, w2, 0.0)
    lf = lf - jnp.max(lf, axis=-1, keepdims=True)
    ef = jnp.exp(lf)
    feat = ef / jnp.sum(ef, axis=-1, keepdims=True)
    a = a_ref[0, 0]
    hfeats = (0.5 * a) * type_ + (0.5 * (1.0 - a)) * feat
    xb = (0.5 * x).astype(jnp.bfloat16)
    hfsum = jnp.sum(hfeats, axis=-1, keepdims=True)
    C = DIMS // NCH
    for n in range(NCH):
        sl = slice(n * C, (n + 1) * C)
        acc = jnp.broadcast_to(hfsum, (xb.shape[0], C))
        for e in range(E):
            z = jnp.dot(xb, w_ref[e][:, sl].astype(jnp.bfloat16), preferred_element_type=jnp.float32)
            acc = acc + jnp.tanh(z + bg_ref[e][sl]) * hfeats[:, e:e + 1]
        o_ref[:, sl] = acc


def kernel(x, W_gates, b_gates, W_f, b_f, W_t, b_t, alpha, num):
    B, S, D = x.shape
    M = B * S
    xf = x.reshape(M, D)
    a = jax.nn.sigmoid(alpha).reshape(1, 1).astype(jnp.float32)
    Wb = W_gates
    Wc = jnp.concatenate([W_t, W_f], axis=1)
    bc = jnp.concatenate([b_t, b_f]).reshape(1, 2 * E)
    hbg = 0.5 * b_gates

    out = pl.pallas_call(
        _body,
        grid=(M // BT,),
        in_specs=[
            pl.BlockSpec((BT, D), lambda t: (t, 0)),
            pl.BlockSpec((E, D, D), lambda t: (0, 0, 0)),
            pl.BlockSpec((E, D), lambda t: (0, 0)),
            pl.BlockSpec((D, 2 * E), lambda t: (0, 0)),
            pl.BlockSpec((1, 2 * E), lambda t: (0, 0)),
            pl.BlockSpec((1, 1), lambda t: (0, 0)),
        ],
        out_specs=pl.BlockSpec((BT, D), lambda t: (t, 0)),
        out_shape=jax.ShapeDtypeStruct((M, D), jnp.float32),
        compiler_params=pltpu.CompilerParams(
            dimension_semantics=("parallel",),
        ),
    )(xf, Wb, hbg, Wc, bc, a)

    return out.reshape(B, S, D)


# speedup vs baseline: 1.1799x; 1.0008x over previous
import jax
import jax.numpy as jnp
from jax.experimental import pallas as pl
from jax.experimental.pallas import tpu as pltpu

DIMS = 1024
E = 8
BT = 1024
NCH = 4


def _body(x_ref, w_ref, bg_ref, wc_ref, bc_ref, a_ref, o_ref):
    x = x_ref[...]
    lc = jnp.dot(x, wc_ref[...], preferred_element_type=jnp.float32) + bc_ref[...]
    lt = lc[:, :E]
    lf = lc[:, E:]
    iota = jax.lax.broadcasted_iota(jnp.int32, lt.shape, 1)
    m1 = jnp.max(lt, axis=-1, keepdims=True)
    i1 = jnp.min(jnp.where(lt == m1, iota, E), axis=-1, keepdims=True)
    masked = jnp.where(iota == i1, -jnp.inf, lt)
    m2 = jnp.max(masked, axis=-1, keepdims=True)
    i2 = jnp.min(jnp.where(masked == m2, iota, E), axis=-1, keepdims=True)
    t = jnp.exp(m2 - m1)
    w1 = 1.0 / (1.0 + t)
    w2 = t / (1.0 + t)
    type_ = jnp.where(iota == i1, w1, 0.0) + jnp.where(iota == i2, w2, 0.0)
    lf = lf - jnp.max(lf, axis=-1, keepdims=True)
    ef = jnp.exp(lf)
    feat = ef / jnp.sum(ef, axis=-1, keepdims=True)
    a = a_ref[0, 0]
    hfeats = (0.5 * a) * type_ + (0.5 * (1.0 - a)) * feat
    xb = (0.5 * x).astype(jnp.bfloat16)
    hfsum = jnp.sum(hfeats, axis=-1, keepdims=True)
    C = DIMS // NCH
    for n in range(NCH):
        sl = slice(n * C, (n + 1) * C)
        acc = jnp.broadcast_to(hfsum, (xb.shape[0], C))
        for e in range(E):
            z = jnp.dot(
                xb,
                w_ref[e][:, sl].astype(jnp.bfloat16),
                preferred_element_type=jnp.float32,
            )
            acc = acc + jnp.tanh(z + bg_ref[e][sl]) * hfeats[:, e:e + 1]
        o_ref[:, sl] = acc


def kernel(x, W_gates, b_gates, W_f, b_f, W_t, b_t, alpha, num):
    B, S, D = x.shape
    M = B * S
    xf = x.reshape(M, D)
    a = jax.nn.sigmoid(alpha).reshape(1, 1).astype(jnp.float32)
    Wc = jnp.concatenate([W_t, W_f], axis=1)
    bc = jnp.concatenate([b_t, b_f]).reshape(1, 2 * E)
    hbg = 0.5 * b_gates

    out = pl.pallas_call(
        _body,
        grid=(M // BT,),
        in_specs=[
            pl.BlockSpec((BT, D), lambda t: (t, 0)),
            pl.BlockSpec((E, D, D), lambda t: (0, 0, 0)),
            pl.BlockSpec((E, D), lambda t: (0, 0)),
            pl.BlockSpec((D, 2 * E), lambda t: (0, 0)),
            pl.BlockSpec((1, 2 * E), lambda t: (0, 0)),
            pl.BlockSpec((1, 1), lambda t: (0, 0)),
        ],
        out_specs=pl.BlockSpec((BT, D), lambda t: (t, 0)),
        out_shape=jax.ShapeDtypeStruct((M, D), jnp.float32),
        compiler_params=pltpu.CompilerParams(
            dimension_semantics=("parallel",),
        ),
    )(xf, W_gates, hbg, Wc, bc, a)

    return out.reshape(B, S, D)
